# baseline (speedup 1.0000x reference)
import functools

import jax
import jax.numpy as jnp
from jax import lax
from jax.experimental import pallas as pl
from jax.experimental.pallas import tpu as pltpu
from jax.experimental.pallas import tpu_sc as plsc

HIDDIM = 128
N_NODES = 10000
N_EDGES = 320000
N_TUPLES = 320000

NC = 2
NS = 16
NW = NC * NS

PA = N_EDGES // NW
C = 128
NFULL = PA // C
TAILA = PA - NFULL * C

PX = 312
CX = 104
XNF = PX // CX
TAILX = N_NODES - NW * PX

K = 6
G = 4

_MESH = plsc.VectorSubcoreMesh(
    core_axis_name="c", subcore_axis_name="s", num_cores=NC, num_subcores=NS
)


def _body(*refs):
  (x_idx, a_idx, x_tab, a_tab,
   out_x, out_a,
   idx_v, idx_x, xtail, xtab_v, atab_v) = refs[:11]
  bufs = refs[11:11 + K]
  gsems = refs[11 + K:11 + 2 * K]
  osems = refs[11 + 2 * K:11 + 3 * K]
  sem_xg, sem_xo = refs[11 + 3 * K:]

  wid = lax.axis_index("s") * NC + lax.axis_index("c")
  is_last = wid == NW - 1

  @pl.when(lax.axis_index("s") == 0)
  def _():
    pltpu.sync_copy(x_tab, xtab_v)
    pltpu.sync_copy(a_tab, atab_v)
  plsc.subcore_barrier()

  pending = {}

  def o_drain(j):
    size = pending.pop(j, None)
    if size is not None:
      pltpu.make_async_copy(bufs[j].at[pl.ds(0, size)],
                            out_x.at[pl.ds(0, size)], osems[j]).wait()

  xbase = wid * PX
  pltpu.sync_copy(x_idx.at[pl.ds(xbase, PX)], idx_v.at[pl.ds(0, PX)])
  for c in range(XNF):
    pltpu.async_copy(xtab_v.at[idx_v.at[pl.ds(c * CX, CX)]],
                     bufs[c].at[pl.ds(0, CX)], gsems[c])
  @pl.when(is_last)
  def _():
    pltpu.sync_copy(x_idx.at[pl.ds(NW * PX, TAILX)], idx_x)
    pltpu.async_copy(xtab_v.at[idx_x], xtail, sem_xg)

  for c in range(XNF):
    pltpu.make_async_copy(xtab_v.at[idx_v.at[pl.ds(0, CX)]],
                          bufs[c].at[pl.ds(0, CX)], gsems[c]).wait()
    pltpu.async_copy(bufs[c].at[pl.ds(0, CX)],
                     out_x.at[pl.ds(xbase + c * CX, CX)], osems[c])
    pending[c] = CX

  @pl.when(is_last)
  def _():
    pltpu.make_async_copy(xtab_v.at[idx_x], xtail, sem_xg).wait()
    pltpu.async_copy(xtail, out_x.at[pl.ds(NW * PX, TAILX)], sem_xo)

  def big_stream(tab, idx_hbm, out_hbm):
    base = wid * PA
    pltpu.sync_copy(idx_hbm.at[pl.ds(base, PA)], idx_v)

    def g_start(off, j, size=C):
      pltpu.async_copy(tab.at[idx_v.at[pl.ds(off, size)]],
                       bufs[j].at[pl.ds(0, size)], gsems[j])

    def g_wait(j, size=C):
      pltpu.make_async_copy(tab.at[idx_v.at[pl.ds(0, size)]],
                            bufs[j].at[pl.ds(0, size)], gsems[j]).wait()

    def o_start(off, j, size=C):
      pltpu.async_copy(bufs[j].at[pl.ds(0, size)],
                       out_hbm.at[pl.ds(base + off, size)], osems[j])

    def o_wait(j, size=C):
      pltpu.make_async_copy(bufs[j].at[pl.ds(0, size)],
                            out_hbm.at[pl.ds(0, size)], osems[j]).wait()

    for c in range(G):
      o_drain(c % K)
      g_start(c * C, c % K)

    for c in range(2):
      cp = c + G
      o_drain(cp % K)
      g_start(cp * C, cp % K)
      g_wait(c % K)
      o_start(c * C, c % K)
      pending[c % K] = C

    def group(gi, carry):
      c0 = 2 + 6 * gi
      for j in range(6):
        jb_c = (2 + j) % K
        jb_p = (2 + j + G) % K
        o_wait(jb_p)
        g_start((c0 + j + G) * C, jb_p)
        g_wait(jb_c)
        o_start((c0 + j) * C, jb_c)
      return carry

    lax.fori_loop(0, (NFULL - G - 2) // K, group, 0)
    pending.clear()
    pending[72 % K] = C
    pending[73 % K] = C

    for c in range(NFULL - G, NFULL):
      if c == NFULL - G:
        o_drain(NFULL % K)
        g_start(NFULL * C, NFULL % K, TAILA)
      g_wait(c % K)
      o_start(c * C, c % K)
      pending[c % K] = C
    g_wait(NFULL % K, TAILA)
    o_start(NFULL * C, NFULL % K, TAILA)
    pending[NFULL % K] = TAILA

  big_stream(atab_v, a_idx, out_a)

  for j in list(pending):
    o_drain(j)

  @pl.when(is_last)
  def _():
    pltpu.make_async_copy(xtail, out_x.at[pl.ds(NW * PX, TAILX)],
                          sem_xo).wait()


_sc_gather = functools.partial(
    pl.kernel,
    out_type=(
        jax.ShapeDtypeStruct((N_NODES, HIDDIM), jnp.float32),
        jax.ShapeDtypeStruct((N_EDGES, HIDDIM), jnp.float32),
    ),
    mesh=_MESH,
    scratch_types=(
        [
            pltpu.VMEM((PA,), jnp.int32),
            pltpu.VMEM((TAILX,), jnp.int32),
            pltpu.VMEM((TAILX, HIDDIM), jnp.float32),
            pltpu.VMEM_SHARED((32, HIDDIM), jnp.float32),
            pltpu.VMEM_SHARED((16, HIDDIM), jnp.float32),
        ]
        + [pltpu.VMEM((C, HIDDIM), jnp.float32)] * K
        + [pltpu.SemaphoreType.DMA] * (2 * K)
        + [pltpu.SemaphoreType.DMA] * 2
    ),
)(_body)


TCB = 16000


def _tc_body(idx_ref, tab_ref, out_ref):
  idx = idx_ref[0, 0, :]
  oh = idx[:, None] == lax.broadcasted_iota(jnp.int32, (TCB, 16), 1)
  out_ref[...] = jnp.dot(oh.astype(jnp.float32), tab_ref[...],
                         preferred_element_type=jnp.float32)


_tc_gather = pl.pallas_call(
    _tc_body,
    grid=(N_TUPLES // TCB,),
    in_specs=[
        pl.BlockSpec((1, 1, TCB), lambda i: (i, 0, 0)),
        pl.BlockSpec((16, HIDDIM), lambda i: (0, 0)),
    ],
    out_specs=pl.BlockSpec((TCB, HIDDIM), lambda i: (i, 0)),
    out_shape=jax.ShapeDtypeStruct((N_TUPLES, HIDDIM), jnp.float32),
)


@jax.jit
def kernel(x, A_values, X_values, x_table, ea_table, tuple_table):
  t_emb = _tc_gather(
      X_values.astype(jnp.int32).reshape(N_TUPLES // TCB, 1, TCB),
      tuple_table)
  x_emb, a_emb = _sc_gather(
      x.astype(jnp.int32),
      A_values.astype(jnp.int32),
      x_table, ea_table)
  return x_emb, a_emb, t_emb


# speedup vs baseline: 1.0020x; 1.0020x over previous
import functools

import jax
import jax.numpy as jnp
from jax import lax
from jax.experimental import pallas as pl
from jax.experimental.pallas import tpu as pltpu
from jax.experimental.pallas import tpu_sc as plsc

HIDDIM = 128
N_NODES = 10000
N_EDGES = 320000
N_TUPLES = 320000

NC = 2
NS = 16
NW = NC * NS

PA = N_EDGES // NW
C = 128
NFULL = PA // C
TAILA = PA - NFULL * C

PX = 312
CX = 104
XNF = PX // CX
TAILX = N_NODES - NW * PX

K = 6
G = 4

_MESH = plsc.VectorSubcoreMesh(
    core_axis_name="c", subcore_axis_name="s", num_cores=NC, num_subcores=NS
)


def _body(*refs):
  (x_idx, a_idx, x_tab, a_tab,
   out_x, out_a,
   idx_v, idx_x, xtail, xtab_v, atab_v) = refs[:11]
  bufs = refs[11:11 + K]
  gsems = refs[11 + K:11 + 2 * K]
  osems = refs[11 + 2 * K:11 + 3 * K]
  sem_xg, sem_xo = refs[11 + 3 * K:]

  wid = lax.axis_index("s") * NC + lax.axis_index("c")
  is_last = wid == NW - 1

  @pl.when(lax.axis_index("s") == 0)
  def _():
    pltpu.sync_copy(x_tab, xtab_v)
    pltpu.sync_copy(a_tab, atab_v)
  plsc.subcore_barrier()

  pending = {}

  def o_drain(j):
    size = pending.pop(j, None)
    if size is not None:
      pltpu.make_async_copy(bufs[j].at[pl.ds(0, size)],
                            out_x.at[pl.ds(0, size)], osems[j]).wait()

  xbase = wid * PX
  pltpu.sync_copy(x_idx.at[pl.ds(xbase, PX)], idx_v.at[pl.ds(0, PX)])
  for c in range(XNF):
    pltpu.async_copy(xtab_v.at[idx_v.at[pl.ds(c * CX, CX)]],
                     bufs[c].at[pl.ds(0, CX)], gsems[c])
  @pl.when(is_last)
  def _():
    pltpu.sync_copy(x_idx.at[pl.ds(NW * PX, TAILX)], idx_x)
    pltpu.async_copy(xtab_v.at[idx_x], xtail, sem_xg)

  for c in range(XNF):
    pltpu.make_async_copy(xtab_v.at[idx_v.at[pl.ds(0, CX)]],
                          bufs[c].at[pl.ds(0, CX)], gsems[c]).wait()
    pltpu.async_copy(bufs[c].at[pl.ds(0, CX)],
                     out_x.at[pl.ds(xbase + c * CX, CX)], osems[c])
    pending[c] = CX

  @pl.when(is_last)
  def _():
    pltpu.make_async_copy(xtab_v.at[idx_x], xtail, sem_xg).wait()
    pltpu.async_copy(xtail, out_x.at[pl.ds(NW * PX, TAILX)], sem_xo)

  def big_stream(tab, idx_hbm, out_hbm):
    base = wid * PA
    pltpu.sync_copy(idx_hbm.at[pl.ds(base, PA)], idx_v)

    def g_start(off, j, size=C):
      pltpu.async_copy(tab.at[idx_v.at[pl.ds(off, size)]],
                       bufs[j].at[pl.ds(0, size)], gsems[j])

    def g_wait(j, size=C):
      pltpu.make_async_copy(tab.at[idx_v.at[pl.ds(0, size)]],
                            bufs[j].at[pl.ds(0, size)], gsems[j]).wait()

    def o_start(off, j, size=C):
      pltpu.async_copy(bufs[j].at[pl.ds(0, size)],
                       out_hbm.at[pl.ds(base + off, size)], osems[j])

    def o_wait(j, size=C):
      pltpu.make_async_copy(bufs[j].at[pl.ds(0, size)],
                            out_hbm.at[pl.ds(0, size)], osems[j]).wait()

    for c in range(G):
      o_drain(c % K)
      g_start(c * C, c % K)

    for c in range(2):
      cp = c + G
      o_drain(cp % K)
      g_start(cp * C, cp % K)
      g_wait(c % K)
      o_start(c * C, c % K)
      pending[c % K] = C

    def group(gi, carry):
      c0 = 2 + 6 * gi
      for j in range(6):
        jb_c = (2 + j) % K
        jb_p = (2 + j + G) % K
        o_wait(jb_p)
        g_start((c0 + j + G) * C, jb_p)
        g_wait(jb_c)
        o_start((c0 + j) * C, jb_c)
      return carry

    lax.fori_loop(0, (NFULL - G - 2) // K, group, 0)
    pending.clear()
    pending[72 % K] = C
    pending[73 % K] = C

    for c in range(NFULL - G, NFULL):
      if c == NFULL - G:
        o_drain(NFULL % K)
        g_start(NFULL * C, NFULL % K, TAILA)
      g_wait(c % K)
      o_start(c * C, c % K)
      pending[c % K] = C
    g_wait(NFULL % K, TAILA)
    o_start(NFULL * C, NFULL % K, TAILA)
    pending[NFULL % K] = TAILA

  big_stream(atab_v, a_idx, out_a)

  for j in list(pending):
    o_drain(j)

  @pl.when(is_last)
  def _():
    pltpu.make_async_copy(xtail, out_x.at[pl.ds(NW * PX, TAILX)],
                          sem_xo).wait()


_sc_gather = functools.partial(
    pl.kernel,
    out_type=(
        jax.ShapeDtypeStruct((N_NODES, HIDDIM), jnp.float32),
        jax.ShapeDtypeStruct((N_EDGES, HIDDIM), jnp.float32),
    ),
    mesh=_MESH,
    scratch_types=(
        [
            pltpu.VMEM((PA,), jnp.int32),
            pltpu.VMEM((TAILX,), jnp.int32),
            pltpu.VMEM((TAILX, HIDDIM), jnp.float32),
            pltpu.VMEM_SHARED((32, HIDDIM), jnp.float32),
            pltpu.VMEM_SHARED((16, HIDDIM), jnp.float32),
        ]
        + [pltpu.VMEM((C, HIDDIM), jnp.float32)] * K
        + [pltpu.SemaphoreType.DMA] * (2 * K)
        + [pltpu.SemaphoreType.DMA] * 2
    ),
)(_body)


TCB = 32000


def _tc_body(idx_ref, tab_ref, out_ref):
  idx = idx_ref[0, 0, :]
  oh = idx[:, None] == lax.broadcasted_iota(jnp.int32, (TCB, 16), 1)
  out_ref[...] = jnp.dot(oh.astype(jnp.float32), tab_ref[...],
                         preferred_element_type=jnp.float32)


_tc_gather = pl.pallas_call(
    _tc_body,
    grid=(N_TUPLES // TCB,),
    in_specs=[
        pl.BlockSpec((1, 1, TCB), lambda i: (i, 0, 0)),
        pl.BlockSpec((16, HIDDIM), lambda i: (0, 0)),
    ],
    out_specs=pl.BlockSpec((TCB, HIDDIM), lambda i: (i, 0)),
    out_shape=jax.ShapeDtypeStruct((N_TUPLES, HIDDIM), jnp.float32),
)


@jax.jit
def kernel(x, A_values, X_values, x_table, ea_table, tuple_table):
  t_emb = _tc_gather(
      X_values.astype(jnp.int32).reshape(N_TUPLES // TCB, 1, TCB),
      tuple_table)
  x_emb, a_emb = _sc_gather(
      x.astype(jnp.int32),
      A_values.astype(jnp.int32),
      x_table, ea_table)
  return x_emb, a_emb, t_emb


# speedup vs baseline: 1.0035x; 1.0015x over previous
import functools

import jax
import jax.numpy as jnp
from jax import lax
from jax.experimental import pallas as pl
from jax.experimental.pallas import tpu as pltpu
from jax.experimental.pallas import tpu_sc as plsc

HIDDIM = 128
N_NODES = 10000
N_EDGES = 320000
N_TUPLES = 320000

NC = 2
NS = 16
NW = NC * NS

PA = N_EDGES // NW
C = 128
NFULL = PA // C
TAILA = PA - NFULL * C

PX = 312
CX = 104
XNF = PX // CX
TAILX = N_NODES - NW * PX

K = 6
G = 4

_MESH = plsc.VectorSubcoreMesh(
    core_axis_name="c", subcore_axis_name="s", num_cores=NC, num_subcores=NS
)


def _body(*refs):
  (x_idx, a_idx, x_tab, a_tab,
   out_x, out_a,
   idx_v, idx_x, xtail, xtab_v, atab_v) = refs[:11]
  bufs = refs[11:11 + K]
  gsems = refs[11 + K:11 + 2 * K]
  osems = refs[11 + 2 * K:11 + 3 * K]
  sem_xg, sem_xo = refs[11 + 3 * K:]

  wid = lax.axis_index("s") * NC + lax.axis_index("c")
  is_last = wid == NW - 1

  @pl.when(lax.axis_index("s") == 0)
  def _():
    pltpu.sync_copy(x_tab, xtab_v)
    pltpu.sync_copy(a_tab, atab_v)
  plsc.subcore_barrier()

  pending = {}

  def o_drain(j):
    size = pending.pop(j, None)
    if size is not None:
      pltpu.make_async_copy(bufs[j].at[pl.ds(0, size)],
                            out_x.at[pl.ds(0, size)], osems[j]).wait()

  xbase = wid * PX
  pltpu.sync_copy(x_idx.at[pl.ds(xbase, PX)], idx_v.at[pl.ds(0, PX)])
  for c in range(XNF):
    pltpu.async_copy(xtab_v.at[idx_v.at[pl.ds(c * CX, CX)]],
                     bufs[c].at[pl.ds(0, CX)], gsems[c])
  @pl.when(is_last)
  def _():
    pltpu.sync_copy(x_idx.at[pl.ds(NW * PX, TAILX)], idx_x)
    pltpu.async_copy(xtab_v.at[idx_x], xtail, sem_xg)

  for c in range(XNF):
    pltpu.make_async_copy(xtab_v.at[idx_v.at[pl.ds(0, CX)]],
                          bufs[c].at[pl.ds(0, CX)], gsems[c]).wait()
    pltpu.async_copy(bufs[c].at[pl.ds(0, CX)],
                     out_x.at[pl.ds(xbase + c * CX, CX)], osems[c])
    pending[c] = CX

  @pl.when(is_last)
  def _():
    pltpu.make_async_copy(xtab_v.at[idx_x], xtail, sem_xg).wait()
    pltpu.async_copy(xtail, out_x.at[pl.ds(NW * PX, TAILX)], sem_xo)

  def big_stream(tab, idx_hbm, out_hbm):
    base = wid * PA
    pltpu.sync_copy(idx_hbm.at[pl.ds(base, PA)], idx_v)

    def g_start(off, j, size=C):
      pltpu.async_copy(tab.at[idx_v.at[pl.ds(off, size)]],
                       bufs[j].at[pl.ds(0, size)], gsems[j])

    def g_wait(j, size=C):
      pltpu.make_async_copy(tab.at[idx_v.at[pl.ds(0, size)]],
                            bufs[j].at[pl.ds(0, size)], gsems[j]).wait()

    def o_start(off, j, size=C):
      pltpu.async_copy(bufs[j].at[pl.ds(0, size)],
                       out_hbm.at[pl.ds(base + off, size)], osems[j])

    def o_wait(j, size=C):
      pltpu.make_async_copy(bufs[j].at[pl.ds(0, size)],
                            out_hbm.at[pl.ds(0, size)], osems[j]).wait()

    for c in range(G):
      o_drain(c % K)
      g_start(c * C, c % K)

    for c in range(2):
      cp = c + G
      o_drain(cp % K)
      g_start(cp * C, cp % K)
      g_wait(c % K)
      o_start(c * C, c % K)
      pending[c % K] = C

    def group(gi, carry):
      c0 = 2 + 6 * gi
      for j in range(6):
        jb_c = (2 + j) % K
        jb_p = (2 + j + G) % K
        o_wait(jb_p)
        g_start((c0 + j + G) * C, jb_p)
        g_wait(jb_c)
        o_start((c0 + j) * C, jb_c)
      return carry

    lax.fori_loop(0, (NFULL - G - 2) // K, group, 0)
    pending.clear()
    pending[72 % K] = C
    pending[73 % K] = C

    for c in range(NFULL - G, NFULL):
      if c == NFULL - G:
        o_drain(NFULL % K)
        g_start(NFULL * C, NFULL % K, TAILA)
      g_wait(c % K)
      o_start(c * C, c % K)
      pending[c % K] = C
    g_wait(NFULL % K, TAILA)
    o_start(NFULL * C, NFULL % K, TAILA)
    pending[NFULL % K] = TAILA

  big_stream(atab_v, a_idx, out_a)

  for j in list(pending):
    o_drain(j)

  @pl.when(is_last)
  def _():
    pltpu.make_async_copy(xtail, out_x.at[pl.ds(NW * PX, TAILX)],
                          sem_xo).wait()


_sc_gather = functools.partial(
    pl.kernel,
    out_type=(
        jax.ShapeDtypeStruct((N_NODES, HIDDIM), jnp.float32),
        jax.ShapeDtypeStruct((N_EDGES, HIDDIM), jnp.float32),
    ),
    mesh=_MESH,
    scratch_types=(
        [
            pltpu.VMEM((PA,), jnp.int32),
            pltpu.VMEM((TAILX,), jnp.int32),
            pltpu.VMEM((TAILX, HIDDIM), jnp.float32),
            pltpu.VMEM_SHARED((32, HIDDIM), jnp.float32),
            pltpu.VMEM_SHARED((16, HIDDIM), jnp.float32),
        ]
        + [pltpu.VMEM((C, HIDDIM), jnp.float32)] * K
        + [pltpu.SemaphoreType.DMA] * (2 * K)
        + [pltpu.SemaphoreType.DMA] * 2
    ),
)(_body)


TCB = 16000


def _tc_body(idx_ref, tab_ref, out_ref):
  idx = idx_ref[0, 0, :]
  oh = idx[:, None] == lax.broadcasted_iota(jnp.int32, (TCB, 16), 1)
  out_ref[...] = jnp.dot(oh.astype(jnp.float32), tab_ref[...],
                         preferred_element_type=jnp.float32)


_tc_gather = pl.pallas_call(
    _tc_body,
    grid=(N_TUPLES // TCB,),
    in_specs=[
        pl.BlockSpec((1, 1, TCB), lambda i: (i, 0, 0)),
        pl.BlockSpec((16, HIDDIM), lambda i: (0, 0)),
    ],
    out_specs=pl.BlockSpec((TCB, HIDDIM), lambda i: (i, 0)),
    out_shape=jax.ShapeDtypeStruct((N_TUPLES, HIDDIM), jnp.float32),
)


@jax.jit
def kernel(x, A_values, X_values, x_table, ea_table, tuple_table):
  t_emb = _tc_gather(
      X_values.astype(jnp.int32).reshape(N_TUPLES // TCB, 1, TCB),
      tuple_table)
  x_emb, a_emb = _sc_gather(
      x.astype(jnp.int32),
      A_values.astype(jnp.int32),
      x_table, ea_table)
  return x_emb, a_emb, t_emb
